# baseline (speedup 1.0000x reference)
import functools

import jax
import jax.numpy as jnp
from jax import lax
from jax.experimental import pallas as pl
from jax.experimental.pallas import tpu as pltpu
from jax.experimental.pallas import tpu_sc as plsc

NC = 2
NS = 16
NW = NC * NS
LANES = 16
NBUF = 8


@functools.lru_cache(maxsize=None)
def _build(rows, vocab, d, seq_len, chunk):
    nchunk_total = rows // chunk
    nchunk = nchunk_total // NW
    rpw = rows // NW
    vregs_per_row = d // LANES
    pos_period = seq_len // chunk
    assert rows % (NW * seq_len) == 0
    assert seq_len % chunk == 0 and pos_period == 2
    assert NBUF % 2 == 0 and nchunk % NBUF == 0
    assert d % LANES == 0 and rows % chunk == 0 and nchunk_total % NW == 0

    mesh = plsc.VectorSubcoreMesh(core_axis_name="c", subcore_axis_name="s")

    @functools.partial(
        pl.kernel,
        mesh=mesh,
        out_type=jax.ShapeDtypeStruct((nchunk_total, chunk, d), jnp.float32),
        compiler_params=pltpu.CompilerParams(use_tc_tiling_on_sc=False),
        scratch_types=[
            pltpu.VMEM((nchunk, chunk), jnp.int32),
            pltpu.VMEM((seq_len, d), jnp.float32),
            pltpu.VMEM((NBUF, chunk, d), jnp.float32),
            [pltpu.SemaphoreType.DMA] * NBUF,
            [pltpu.SemaphoreType.DMA] * NBUF,
        ],
    )
    def embed(table_hbm, idx_hbm, pos_hbm, out_hbm, idx_v, pos_v, rows_v,
              gsems, wsems):
        wid = lax.axis_index("s") * NC + lax.axis_index("c")
        chunk_base = wid * nchunk

        pltpu.sync_copy(idx_hbm.at[pl.ds(chunk_base, nchunk)], idx_v)
        pltpu.sync_copy(pos_hbm, pos_v)

        for b in range(NBUF):
            pltpu.async_copy(table_hbm.at[idx_v.at[b]], rows_v.at[b], gsems[b])

        @pl.loop(0, nchunk, step=NBUF)
        def chunk_loop(j0):
            for b in range(NBUF):
                j = j0 + b
                buf = rows_v.at[b]

                pltpu.make_async_copy(table_hbm.at[idx_v.at[j]],
                                      buf, gsems[b]).wait()

                pos_off = (b % 2) * chunk

                @pl.loop(0, chunk)
                def add_loop(r):
                    for c in range(vregs_per_row):
                        sl = pl.ds(c * LANES, LANES)
                        buf[r, sl] = buf[r, sl] + pos_v[pos_off + r, sl]

                pltpu.async_copy(buf, out_hbm.at[chunk_base + j], wsems[b])

                @pl.when(j + NBUF < nchunk)
                def _():
                    pltpu.make_async_copy(buf, out_hbm.at[chunk_base + j],
                                          wsems[b]).wait()
                    pltpu.async_copy(table_hbm.at[idx_v.at[j + NBUF]],
                                     rows_v.at[b], gsems[b])

        for b in range(NBUF):
            pltpu.make_async_copy(rows_v.at[b],
                                  out_hbm.at[chunk_base + nchunk - NBUF + b],
                                  wsems[b]).wait()

    return embed


def kernel(inputs, word_table, pos_table):
    batch, seq_len = inputs.shape
    vocab, d = word_table.shape
    rows = batch * seq_len
    chunk = 100

    embed = _build(rows, vocab, d, seq_len, chunk)
    idx2d = inputs.reshape(rows // chunk, chunk)
    out = embed(word_table, idx2d, pos_table)
    return out.reshape(batch, seq_len, d)


# speedup vs baseline: 1.0074x; 1.0074x over previous
import functools

import jax
import jax.numpy as jnp
from jax import lax
from jax.experimental import pallas as pl
from jax.experimental.pallas import tpu as pltpu
from jax.experimental.pallas import tpu_sc as plsc

NC = 2
NS = 16
NW = NC * NS
LANES = 16
NBUF = 4


@functools.lru_cache(maxsize=None)
def _build(rows, vocab, d, seq_len, chunk):
    nchunk_total = rows // chunk
    nchunk = nchunk_total // NW
    rpw = rows // NW
    vregs_per_row = d // LANES
    assert rows % (NW * seq_len) == 0
    assert chunk % seq_len == 0
    assert nchunk % NBUF == 0
    assert d % LANES == 0 and rows % chunk == 0 and nchunk_total % NW == 0

    mesh = plsc.VectorSubcoreMesh(core_axis_name="c", subcore_axis_name="s")

    @functools.partial(
        pl.kernel,
        mesh=mesh,
        out_type=jax.ShapeDtypeStruct((nchunk_total, chunk, d), jnp.float32),
        compiler_params=pltpu.CompilerParams(use_tc_tiling_on_sc=False),
        scratch_types=[
            pltpu.VMEM((nchunk, chunk), jnp.int32),
            pltpu.VMEM((seq_len, d), jnp.float32),
            pltpu.VMEM((NBUF, chunk, d), jnp.float32),
            [pltpu.SemaphoreType.DMA] * NBUF,
            [pltpu.SemaphoreType.DMA] * NBUF,
        ],
    )
    def embed(table_hbm, idx_hbm, pos_hbm, out_hbm, idx_v, pos_v, rows_v,
              gsems, wsems):
        wid = lax.axis_index("s") * NC + lax.axis_index("c")
        chunk_base = wid * nchunk

        pltpu.sync_copy(idx_hbm.at[pl.ds(chunk_base, nchunk)], idx_v)
        pltpu.sync_copy(pos_hbm, pos_v)

        for b in range(NBUF):
            pltpu.async_copy(table_hbm.at[idx_v.at[b]], rows_v.at[b], gsems[b])

        @pl.loop(0, nchunk, step=NBUF)
        def chunk_loop(j0):
            for b in range(NBUF):
                j = j0 + b
                buf = rows_v.at[b]

                pltpu.make_async_copy(table_hbm.at[idx_v.at[j]],
                                      buf, gsems[b]).wait()

                @pl.loop(0, chunk)
                def add_loop(r):
                    pr = lax.rem(r, seq_len)
                    for c in range(vregs_per_row):
                        sl = pl.ds(c * LANES, LANES)
                        buf[r, sl] = buf[r, sl] + pos_v[pr, sl]

                pltpu.async_copy(buf, out_hbm.at[chunk_base + j], wsems[b])

                @pl.when(j + NBUF < nchunk)
                def _():
                    pltpu.make_async_copy(buf, out_hbm.at[chunk_base + j],
                                          wsems[b]).wait()
                    pltpu.async_copy(table_hbm.at[idx_v.at[j + NBUF]],
                                     rows_v.at[b], gsems[b])

        for b in range(NBUF):
            pltpu.make_async_copy(rows_v.at[b],
                                  out_hbm.at[chunk_base + nchunk - NBUF + b],
                                  wsems[b]).wait()

    return embed


def kernel(inputs, word_table, pos_table):
    batch, seq_len = inputs.shape
    vocab, d = word_table.shape
    rows = batch * seq_len
    chunk = 200

    embed = _build(rows, vocab, d, seq_len, chunk)
    idx2d = inputs.reshape(rows // chunk, chunk)
    out = embed(word_table, idx2d, pos_table)
    return out.reshape(batch, seq_len, d)
